# baseline (speedup 1.0000x reference)
import functools

import jax
import jax.numpy as jnp
from jax import lax
from jax.experimental import pallas as pl
from jax.experimental.pallas import tpu as pltpu
from jax.experimental.pallas import tpu_sc as plsc

NC = 2
NS = 16
NW = NC * NS
BCH = 16
NGRP = 8


@functools.partial(jax.jit, static_argnums=(2, 3, 4))
def _sc_gather(idx_flat, table, b, l, d):
    ct = d // 8
    nplane = l * ct
    per_w_b = b // NW
    chunk = BCH * l
    nchunk = per_w_b // BCH
    npair = nchunk // 2
    gpl = nplane // NGRP
    assert nchunk % 2 == 0 and nchunk >= 4 and nplane % NGRP == 0
    assert BCH == 16 and d % 8 == 0 and b % 128 == 0
    mesh = plsc.VectorSubcoreMesh(core_axis_name="c", subcore_axis_name="s")

    @functools.partial(
        pl.kernel,
        out_type=jax.ShapeDtypeStruct((nplane, b // 128, 8, 128), jnp.float32),
        mesh=mesh,
        scratch_types=[
            pltpu.VMEM((chunk,), jnp.int32),
            pltpu.VMEM((chunk,), jnp.int32),
            pltpu.VMEM((chunk, d), jnp.float32),
            pltpu.VMEM((chunk, d), jnp.float32),
            pltpu.VMEM((gpl, 8, BCH), jnp.float32),
            pltpu.VMEM((gpl, 8, BCH), jnp.float32),
            pltpu.SemaphoreType.DMA,
            pltpu.SemaphoreType.DMA,
            pltpu.SemaphoreType.DMA,
        ],
        compiler_params=pltpu.CompilerParams(use_tc_tiling_on_sc=False, needs_layout_passes=False),
    )
    def k(table_hbm, idx_hbm, out_hbm, idx0, idx1, rows0, rows1, t0, t1,
          g0, g1, wsem):
        wid = lax.axis_index("s") * NC + lax.axis_index("c")
        base = wid * per_w_b * l
        bbase = wid * per_w_b
        lane = lax.iota(jnp.int32, BCH)
        row_iota = lane * l

        def idx_in(c, dst):
            pltpu.sync_copy(idx_hbm.at[pl.ds(base + c * chunk, chunk)], dst)

        def wr_cp(c, g, tbuf):
            b0 = bbase + c * BCH
            tb = b0 // 128
            br0 = b0 % 128
            return pltpu.make_async_copy(
                tbuf,
                out_hbm.at[pl.ds(g * gpl, gpl), tb, :, pl.ds(br0, BCH)],
                wsem,
            )

        def transpose_write(c, src):
            tbufs = (t0, t1)
            for g in range(NGRP):
                tbuf = tbufs[g % 2]
                if g >= 2:
                    wr_cp(c, g - 2, tbuf).wait()

                @pl.loop(0, gpl)
                def _pl(p):
                    pg = g * gpl + p
                    ll = pg // ct
                    tc = pg % ct
                    row_idx = row_iota + ll
                    for cr in range(8):
                        col = jnp.full((BCH,), tc * 8 + cr, jnp.int32)
                        vec = plsc.load_gather(src, [row_idx, col])
                        tbuf[p, cr, :] = vec

                wr_cp(c, g, tbuf).start()
            wr_cp(c, NGRP - 2, t0 if (NGRP - 2) % 2 == 0 else t1).wait()
            wr_cp(c, NGRP - 1, t0 if (NGRP - 1) % 2 == 0 else t1).wait()

        idx_in(0, idx0)
        pltpu.async_copy(table_hbm.at[idx0], rows0, g0)

        @pl.loop(0, npair - 1)
        def _body(pp):
            c = 2 * pp
            idx_in(c + 1, idx1)
            pltpu.make_async_copy(table_hbm.at[idx0], rows0, g0).wait()
            pltpu.async_copy(table_hbm.at[idx1], rows1, g1)
            transpose_write(c, rows0)
            idx_in(c + 2, idx0)
            pltpu.make_async_copy(table_hbm.at[idx1], rows1, g1).wait()
            pltpu.async_copy(table_hbm.at[idx0], rows0, g0)
            transpose_write(c + 1, rows1)

        c = nchunk - 2
        idx_in(c + 1, idx1)
        pltpu.make_async_copy(table_hbm.at[idx0], rows0, g0).wait()
        pltpu.async_copy(table_hbm.at[idx1], rows1, g1)
        transpose_write(c, rows0)
        pltpu.make_async_copy(table_hbm.at[idx1], rows1, g1).wait()
        transpose_write(c + 1, rows1)

    return k(table, idx_flat)


def kernel(indices, table):
    b, l = indices.shape
    d = table.shape[1]
    idx_flat = indices.reshape(-1).astype(jnp.int32)
    out4 = _sc_gather(idx_flat, table, b, l, d)
    y = out4.reshape(l, d // 8, b // 128, 8, 128)
    return y.transpose(2, 4, 0, 1, 3).reshape(b, l, d)


# speedup vs baseline: 2.0251x; 2.0251x over previous
import functools

import jax
import jax.numpy as jnp
from jax import lax
from jax.experimental import pallas as pl
from jax.experimental.pallas import tpu as pltpu
from jax.experimental.pallas import tpu_sc as plsc

NC = 2
NS = 16
NW = NC * NS
BCH = 16


@functools.partial(jax.jit, static_argnums=(2, 3, 4))
def _sc_gather(idx_flat, table, b, l, d):
    ct = d // 8
    nplane = l * ct
    per_w_b = b // NW
    chunk = BCH * l
    nchunk = per_w_b // BCH
    npair = nchunk // 2
    QL = l // 4
    QP = QL * ct
    assert nchunk % 2 == 0 and nchunk >= 4 and l % 4 == 0
    assert BCH == 16 and d == 32 and b % 128 == 0
    mesh = plsc.VectorSubcoreMesh(core_axis_name="c", subcore_axis_name="s")

    @functools.partial(
        pl.kernel,
        out_type=jax.ShapeDtypeStruct((nplane, b // 128, 8, 128), jnp.float32),
        mesh=mesh,
        scratch_types=[
            pltpu.VMEM((chunk,), jnp.int32),
            pltpu.VMEM((chunk,), jnp.int32),
            pltpu.VMEM((chunk, d), jnp.float32),
            pltpu.VMEM((chunk, d), jnp.float32),
            pltpu.VMEM((QP, 8, 17), jnp.float32),
            pltpu.SemaphoreType.DMA,
            pltpu.SemaphoreType.DMA,
        ],
        compiler_params=pltpu.CompilerParams(use_tc_tiling_on_sc=False, needs_layout_passes=False),
    )
    def k(table_hbm, idx_hbm, out_hbm, idx0, idx1, rows0, rows1, t0, g0, g1):
        wid = lax.axis_index("s") * NC + lax.axis_index("c")
        base = wid * per_w_b * l
        bbase = wid * per_w_b
        lane = lax.iota(jnp.int32, BCH)
        row_iota = lane * l

        def idx_in(c, dst):
            pltpu.sync_copy(idx_hbm.at[pl.ds(base + c * chunk, chunk)], dst)

        chi = lane >> 3
        clo = lane & 7

        def transpose_write(c, src):
            b0 = bbase + c * BCH
            tb = b0 // 128
            br0 = b0 % 128
            for q in range(4):

                @pl.loop(0, QL)
                def _tw(ll):
                    lg = q * QL + ll
                    i0a = ll * ct + chi
                    i0b = i0a + (16 // 8)
                    for bp in range(BCH):
                        r = bp * l + lg
                        va = src[r, pl.ds(0, 16)]
                        vb = src[r, pl.ds(16, 16)]
                        i2 = jnp.full((BCH,), bp, jnp.int32)
                        plsc.store_scatter(t0, [i0a, clo, i2], va)
                        plsc.store_scatter(t0, [i0b, clo, i2], vb)

                pltpu.sync_copy(
                    t0.at[:, :, pl.ds(0, BCH)],
                    out_hbm.at[pl.ds(q * QP, QP), tb, :, pl.ds(br0, BCH)],
                )

        idx_in(0, idx0)
        pltpu.async_copy(table_hbm.at[idx0], rows0, g0)

        @pl.loop(0, npair - 1)
        def _body(pp):
            c = 2 * pp
            idx_in(c + 1, idx1)
            pltpu.make_async_copy(table_hbm.at[idx0], rows0, g0).wait()
            pltpu.async_copy(table_hbm.at[idx1], rows1, g1)
            transpose_write(c, rows0)
            idx_in(c + 2, idx0)
            pltpu.make_async_copy(table_hbm.at[idx1], rows1, g1).wait()
            pltpu.async_copy(table_hbm.at[idx0], rows0, g0)
            transpose_write(c + 1, rows1)

        c = nchunk - 2
        idx_in(c + 1, idx1)
        pltpu.make_async_copy(table_hbm.at[idx0], rows0, g0).wait()
        pltpu.async_copy(table_hbm.at[idx1], rows1, g1)
        transpose_write(c, rows0)
        pltpu.make_async_copy(table_hbm.at[idx1], rows1, g1).wait()
        transpose_write(c + 1, rows1)

    return k(table, idx_flat)


def kernel(indices, table):
    b, l = indices.shape
    d = table.shape[1]
    idx_flat = indices.reshape(-1).astype(jnp.int32)
    out4 = _sc_gather(idx_flat, table, b, l, d)
    y = out4.reshape(l, d // 8, b // 128, 8, 128)
    return y.transpose(2, 4, 0, 1, 3).reshape(b, l, d)
